# baseline (speedup 1.0000x reference)
import functools

import jax
import jax.numpy as jnp
from jax import lax
from jax.experimental import pallas as pl
from jax.experimental.pallas import tpu as pltpu
from jax.experimental.pallas import tpu_sc as plsc

N = 10000
E = 320000
D = 128
DZ = 64

NSC = 2
NTILES = 16
NW = NSC * NTILES

STREAM = 128
NSTREAMS = 2560
EPAD = NSTREAMS * STREAM
SPT_AGG = NSTREAMS // NW
SPT_DEG = NSTREAMS // NTILES
ACC_ROWS = 10240
RPT = ACC_ROWS // NTILES
GARBAGE = N
IDXCHUNK = 40


def _rsqrt16(d):
    i = lax.bitcast_convert_type(d, jnp.int32)
    i = jnp.int32(0x5F3759DF) - lax.shift_right_logical(i, jnp.int32(1))
    y = lax.bitcast_convert_type(i, jnp.float32)
    for _ in range(3):
        y = y * (1.5 - 0.5 * d * y * y)
    return y


def _sc_deg_body(ei_hbm, degp_hbm, deg_acc, zb, ones_v, dstbuf, dsem):
    c = lax.axis_index("c")
    s = lax.axis_index("s")
    w = c * NTILES + s

    for k in range(RPT // 16):
        zb[pl.ds(16 * k, 16)] = jnp.zeros((16,), jnp.float32)
    pltpu.sync_copy(zb, deg_acc.at[pl.ds(RPT * s, RPT)])
    for k in range(STREAM // 16):
        ones_v[pl.ds(16 * k, 16)] = jnp.ones((16,), jnp.float32)
    plsc.subcore_barrier()

    pltpu.sync_copy(ei_hbm.at[1, pl.ds(SPT_AGG * w, SPT_AGG)], dstbuf)
    WIN = 32

    @pl.loop(0, SPT_AGG)
    def _deg(j):
        pltpu.async_copy(ones_v, deg_acc.at[dstbuf.at[j]], dsem, add=True)

        @pl.when(j >= WIN)
        def _():
            pltpu.make_async_copy(ones_v, deg_acc.at[dstbuf.at[0]],
                                  dsem).wait()

    @pl.loop(0, WIN)
    def _drain(j):
        pltpu.make_async_copy(ones_v, deg_acc.at[dstbuf.at[0]], dsem).wait()

    plsc.subcore_barrier()
    pltpu.sync_copy(deg_acc.at[pl.ds(RPT * s, RPT)],
                    degp_hbm.at[c, pl.ds(RPT * s, RPT)])


_sc_deg = functools.partial(
    pl.kernel,
    out_type=jax.ShapeDtypeStruct((NSC, ACC_ROWS), jnp.float32),
    mesh=plsc.VectorSubcoreMesh(core_axis_name="c", subcore_axis_name="s"),
    scratch_types=[
        pltpu.VMEM_SHARED((ACC_ROWS,), jnp.float32),
        pltpu.VMEM((RPT,), jnp.float32),
        pltpu.VMEM((STREAM,), jnp.float32),
        pltpu.VMEM((SPT_AGG, STREAM), jnp.int32),
        pltpu.SemaphoreType.DMA,
    ],
)(_sc_deg_body)


def _sc_prep_body(x_hbm, degp_hbm, dinv_hbm, xp_hbm,
                  degbuf, deg2buf, dinvbuf, xbuf):
    c = lax.axis_index("c")
    s = lax.axis_index("s")

    pltpu.sync_copy(degp_hbm.at[0, pl.ds(RPT * s, RPT)], degbuf)
    pltpu.sync_copy(degp_hbm.at[1, pl.ds(RPT * s, RPT)], deg2buf)
    for k in range(RPT // 16):
        dvec = degbuf[pl.ds(16 * k, 16)] + deg2buf[pl.ds(16 * k, 16)] + 1.0
        dinvbuf[pl.ds(16 * k, 16)] = _rsqrt16(dvec)

    @pl.when(jnp.logical_and(c == 0, s < NTILES - 1))
    def _():
        pltpu.sync_copy(dinvbuf.at[pl.ds(0, RPT)],
                        dinv_hbm.at[pl.ds(RPT * s, RPT)])

    @pl.when(jnp.logical_and(c == 0, s == NTILES - 1))
    def _():
        tail = N - RPT * (NTILES - 1)
        pltpu.sync_copy(dinvbuf.at[pl.ds(0, tail)],
                        dinv_hbm.at[pl.ds(RPT * (NTILES - 1), tail)])

    is_writer = jnp.logical_or(jnp.logical_and(c == 0, s < NTILES // 2),
                               jnp.logical_and(c == 1, s >= NTILES // 2))
    nrows = jnp.minimum(RPT, N - RPT * s)
    nchunks = nrows // 16

    @pl.when(is_writer)
    def _():
        @pl.loop(0, RPT // 16)
        def _chunk(i):
            @pl.when(i < nchunks)
            def _():
                row0 = RPT * s + 16 * i
                pltpu.sync_copy(x_hbm.at[pl.ds(row0, 16)], xbuf)
                v16 = dinvbuf[pl.ds(16 * i, 16)]
                for r in range(16):
                    dv = jnp.full((16,), v16[r], jnp.float32)
                    for jj in range(D // 16):
                        sl = pl.ds(16 * jj, 16)
                        xbuf[r, sl] = xbuf[r, sl] * dv
                pltpu.sync_copy(xbuf, xp_hbm.at[pl.ds(row0, 16)])


_sc_prep = functools.partial(
    pl.kernel,
    out_type=(jax.ShapeDtypeStruct((N,), jnp.float32),
              jax.ShapeDtypeStruct((N, D), jnp.float32)),
    mesh=plsc.VectorSubcoreMesh(core_axis_name="c", subcore_axis_name="s"),
    scratch_types=[
        pltpu.VMEM((RPT,), jnp.float32),
        pltpu.VMEM((RPT,), jnp.float32),
        pltpu.VMEM((RPT + 16,), jnp.float32),
        pltpu.VMEM((16, D), jnp.float32),
    ],
)(_sc_prep_body)


def _sc_agg_body(table_hbm, ei_hbm, zrows_hbm, part_hbm,
                 acc, srcbuf, dstbuf, rows, sem0, sem1):
    c = lax.axis_index("c")
    s = lax.axis_index("s")
    w = c * NTILES + s

    pltpu.sync_copy(zrows_hbm, acc.at[pl.ds(RPT * s, RPT)])
    plsc.subcore_barrier()

    rows0 = rows.at[0]
    rows1 = rows.at[1]
    for k in range(SPT_AGG // IDXCHUNK):
        base = SPT_AGG * w + IDXCHUNK * k
        pltpu.sync_copy(ei_hbm.at[0, pl.ds(base, IDXCHUNK)], srcbuf)
        pltpu.sync_copy(ei_hbm.at[1, pl.ds(base, IDXCHUNK)], dstbuf)
        pltpu.async_copy(table_hbm.at[srcbuf.at[0]], rows0, sem0)

        @pl.loop(0, IDXCHUNK, step=2)
        def _edge(j):
            pltpu.async_copy(table_hbm.at[srcbuf.at[j + 1]], rows1, sem1)
            pltpu.make_async_copy(table_hbm.at[srcbuf.at[j]], rows0,
                                  sem0).wait()
            pltpu.sync_copy(rows0, acc.at[dstbuf.at[j]], add=True)

            @pl.when(j + 2 < IDXCHUNK)
            def _():
                pltpu.async_copy(table_hbm.at[srcbuf.at[j + 2]], rows0, sem0)

            pltpu.make_async_copy(table_hbm.at[srcbuf.at[j + 1]], rows1,
                                  sem1).wait()
            pltpu.sync_copy(rows1, acc.at[dstbuf.at[j + 1]], add=True)

    plsc.subcore_barrier()

    @pl.when(s < NTILES - 1)
    def _():
        pltpu.sync_copy(acc.at[pl.ds(RPT * s, RPT)],
                        part_hbm.at[c, pl.ds(RPT * s, RPT)])

    @pl.when(s == NTILES - 1)
    def _():
        tail = N - RPT * (NTILES - 1)
        pltpu.sync_copy(acc.at[pl.ds(RPT * (NTILES - 1), tail)],
                        part_hbm.at[c, pl.ds(RPT * (NTILES - 1), tail)])


_sc_agg = functools.partial(
    pl.kernel,
    out_type=jax.ShapeDtypeStruct((NSC, N, D), jnp.float32),
    mesh=plsc.VectorSubcoreMesh(core_axis_name="c", subcore_axis_name="s"),
    scratch_types=[
        pltpu.VMEM_SHARED((ACC_ROWS, D), jnp.float32),
        pltpu.VMEM((IDXCHUNK, STREAM), jnp.int32),
        pltpu.VMEM((IDXCHUNK, STREAM), jnp.int32),
        pltpu.VMEM((2, STREAM, D), jnp.float32),
        pltpu.SemaphoreType.DMA,
        pltpu.SemaphoreType.DMA,
    ],
)(_sc_agg_body)


ROWB = 1000


def _tc_hidden_body(p01, xp, dinv, w1, b1, o):
    p = p01[...]
    t = dinv[...] * (p[0] + p[1] + xp[...])
    h = jnp.dot(t, w1[...], preferred_element_type=jnp.float32) + b1[...]
    o[...] = dinv[...] * jnp.maximum(h, 0.0)


def _tc_hidden(p01, xp, dinv, w1, b1):
    grid = (N // ROWB,)
    row_spec = pl.BlockSpec((ROWB, D), lambda i: (i, 0))
    return pl.pallas_call(
        _tc_hidden_body,
        grid=grid,
        in_specs=[pl.BlockSpec((2, ROWB, D), lambda i: (0, i, 0)),
                  row_spec,
                  pl.BlockSpec((ROWB, 1), lambda i: (i, 0)),
                  pl.BlockSpec((D, D), lambda i: (0, 0)),
                  pl.BlockSpec((1, D), lambda i: (0, 0))],
        out_specs=row_spec,
        out_shape=jax.ShapeDtypeStruct((N, D), jnp.float32),
    )(p01, xp, dinv, w1, b1)


def _tc_heads_body(q01, hp, dinv, wmu, bmu, wlv, blv, eps, z, mu, lv):
    q = q01[...]
    a = dinv[...] * (q[0] + q[1] + hp[...])
    m = jnp.dot(a, wmu[...], preferred_element_type=jnp.float32) + bmu[...]
    v = jnp.dot(a, wlv[...], preferred_element_type=jnp.float32) + blv[...]
    mu[...] = m
    lv[...] = v
    z[...] = m + jnp.exp(0.5 * v) * eps[...]


def _tc_heads(q01, hp, dinv, wmu, bmu, wlv, blv, eps):
    grid = (N // ROWB,)
    row_spec = pl.BlockSpec((ROWB, D), lambda i: (i, 0))
    z_spec = pl.BlockSpec((ROWB, DZ), lambda i: (i, 0))
    w_spec = pl.BlockSpec((D, DZ), lambda i: (0, 0))
    b_spec = pl.BlockSpec((1, DZ), lambda i: (0, 0))
    zshape = jax.ShapeDtypeStruct((N, DZ), jnp.float32)
    return pl.pallas_call(
        _tc_heads_body,
        grid=grid,
        in_specs=[pl.BlockSpec((2, ROWB, D), lambda i: (0, i, 0)),
                  row_spec,
                  pl.BlockSpec((ROWB, 1), lambda i: (i, 0)),
                  w_spec, b_spec, w_spec, b_spec, z_spec],
        out_specs=(z_spec, z_spec, z_spec),
        out_shape=(zshape, zshape, zshape),
    )(q01, hp, dinv, wmu, bmu, wlv, blv, eps)


def kernel(x, edge_index, W1, b1, Wmu, bmu, Wlv, blv):
    pad = EPAD - E
    src_pad = jnp.arange(pad, dtype=jnp.int32)
    dst_pad = N + jnp.broadcast_to(
        jnp.arange(ACC_ROWS - N, dtype=jnp.int32),
        (pad // (ACC_ROWS - N), ACC_ROWS - N)).reshape(-1)
    ei3 = jnp.concatenate(
        [edge_index.astype(jnp.int32),
         jnp.stack([src_pad, dst_pad])], axis=1).reshape(2, NSTREAMS, STREAM)

    degp = _sc_deg(ei3)
    dinv, xp = _sc_prep(x, degp)
    dinv2d = dinv.reshape(N, 1)
    zrows = jnp.zeros((RPT, D), jnp.float32)

    p = _sc_agg(xp, ei3, zrows)
    hp = _tc_hidden(p, xp, dinv2d, W1, b1.reshape(1, D))

    q = _sc_agg(hp, ei3, zrows)
    eps = jax.random.normal(jax.random.key(42), (N, DZ), jnp.float32)
    z, mu, lv = _tc_heads(q, hp, dinv2d,
                          Wmu, bmu.reshape(1, DZ),
                          Wlv, blv.reshape(1, DZ), eps)
    return z, mu, lv


# speedup vs baseline: 1.0501x; 1.0501x over previous
import functools

import jax
import jax.numpy as jnp
from jax import lax
from jax.experimental import pallas as pl
from jax.experimental.pallas import tpu as pltpu
from jax.experimental.pallas import tpu_sc as plsc

N = 10000
E = 320000
D = 128
DZ = 64

NSC = 2
NTILES = 16
NW = NSC * NTILES

STREAM = 128
NSTREAMS = 2560
EPAD = NSTREAMS * STREAM
SPT_AGG = NSTREAMS // NW
SPT_DEG = NSTREAMS // NTILES
ACC_ROWS = 10240
RPT = ACC_ROWS // NTILES
GARBAGE = N
IDXCHUNK = 40


def _rsqrt16(d):
    i = lax.bitcast_convert_type(d, jnp.int32)
    i = jnp.int32(0x5F3759DF) - lax.shift_right_logical(i, jnp.int32(1))
    y = lax.bitcast_convert_type(i, jnp.float32)
    for _ in range(3):
        y = y * (1.5 - 0.5 * d * y * y)
    return y


def _sc_deg_body(ei_hbm, degp_hbm, deg_acc, zb, ones_v, dstbuf, dsem):
    c = lax.axis_index("c")
    s = lax.axis_index("s")
    w = c * NTILES + s

    for k in range(RPT // 16):
        zb[pl.ds(16 * k, 16)] = jnp.zeros((16,), jnp.float32)
    pltpu.sync_copy(zb, deg_acc.at[pl.ds(RPT * s, RPT)])
    for k in range(STREAM // 16):
        ones_v[pl.ds(16 * k, 16)] = jnp.ones((16,), jnp.float32)
    plsc.subcore_barrier()

    pltpu.sync_copy(ei_hbm.at[1, pl.ds(SPT_AGG * w, SPT_AGG)], dstbuf)
    WIN = 32

    @pl.loop(0, SPT_AGG)
    def _deg(j):
        pltpu.async_copy(ones_v, deg_acc.at[dstbuf.at[j]], dsem, add=True)

        @pl.when(j >= WIN)
        def _():
            pltpu.make_async_copy(ones_v, deg_acc.at[dstbuf.at[0]],
                                  dsem).wait()

    @pl.loop(0, WIN)
    def _drain(j):
        pltpu.make_async_copy(ones_v, deg_acc.at[dstbuf.at[0]], dsem).wait()

    plsc.subcore_barrier()
    pltpu.sync_copy(deg_acc.at[pl.ds(RPT * s, RPT)],
                    degp_hbm.at[pl.ds(c * ACC_ROWS + RPT * s, RPT)])


_sc_deg = functools.partial(
    pl.kernel,
    out_type=jax.ShapeDtypeStruct((NSC * ACC_ROWS,), jnp.float32),
    mesh=plsc.VectorSubcoreMesh(core_axis_name="c", subcore_axis_name="s"),
    scratch_types=[
        pltpu.VMEM_SHARED((ACC_ROWS,), jnp.float32),
        pltpu.VMEM((RPT,), jnp.float32),
        pltpu.VMEM((STREAM,), jnp.float32),
        pltpu.VMEM((SPT_AGG, STREAM), jnp.int32),
        pltpu.SemaphoreType.DMA,
    ],
)(_sc_deg_body)


RPW = ACC_ROWS // NW


def _sc_prep_body(x_hbm, degp_hbm, dinv_hbm, xp_hbm,
                  degbuf, deg2buf, dinvbuf, xbuf):
    c = lax.axis_index("c")
    s = lax.axis_index("s")
    w = c * NTILES + s

    pltpu.sync_copy(degp_hbm.at[pl.ds(RPW * w, RPW)], degbuf)
    pltpu.sync_copy(degp_hbm.at[pl.ds(ACC_ROWS + RPW * w, RPW)], deg2buf)
    for k in range(RPW // 16):
        dvec = degbuf[pl.ds(16 * k, 16)] + deg2buf[pl.ds(16 * k, 16)] + 1.0
        dinvbuf[pl.ds(16 * k, 16)] = _rsqrt16(dvec)

    nrows = jnp.minimum(RPW, jnp.maximum(N - RPW * w, 0))

    @pl.when(nrows == RPW)
    def _():
        pltpu.sync_copy(dinvbuf.at[pl.ds(0, RPW)],
                        dinv_hbm.at[pl.ds(RPW * w, RPW)])

    @pl.when(jnp.logical_and(nrows > 0, nrows < RPW))
    def _():
        tail = N - RPW * (NW - 1)
        pltpu.sync_copy(dinvbuf.at[pl.ds(0, tail)],
                        dinv_hbm.at[pl.ds(RPW * (NW - 1), tail)])

    nchunks = nrows // 16

    @pl.loop(0, RPW // 16)
    def _chunk(i):
        @pl.when(i < nchunks)
        def _():
            row0 = RPW * w + 16 * i
            pltpu.sync_copy(x_hbm.at[pl.ds(row0, 16)], xbuf)
            v16 = dinvbuf[pl.ds(16 * i, 16)]
            for r in range(16):
                dv = jnp.full((16,), v16[r], jnp.float32)
                for jj in range(D // 16):
                    sl = pl.ds(16 * jj, 16)
                    xbuf[r, sl] = xbuf[r, sl] * dv
            pltpu.sync_copy(xbuf, xp_hbm.at[pl.ds(row0, 16)])


_sc_prep = functools.partial(
    pl.kernel,
    out_type=(jax.ShapeDtypeStruct((N,), jnp.float32),
              jax.ShapeDtypeStruct((N, D), jnp.float32)),
    mesh=plsc.VectorSubcoreMesh(core_axis_name="c", subcore_axis_name="s"),
    scratch_types=[
        pltpu.VMEM((RPW,), jnp.float32),
        pltpu.VMEM((RPW,), jnp.float32),
        pltpu.VMEM((RPW + 16,), jnp.float32),
        pltpu.VMEM((16, D), jnp.float32),
    ],
)(_sc_prep_body)


def _sc_agg_body(table_hbm, ei_hbm, zrows_hbm, part_hbm,
                 acc, srcbuf, dstbuf, rows, sem0, sem1):
    c = lax.axis_index("c")
    s = lax.axis_index("s")
    w = c * NTILES + s

    pltpu.sync_copy(zrows_hbm, acc.at[pl.ds(RPT * s, RPT)])
    plsc.subcore_barrier()

    rows0 = rows.at[0]
    rows1 = rows.at[1]
    for k in range(SPT_AGG // IDXCHUNK):
        base = SPT_AGG * w + IDXCHUNK * k
        pltpu.sync_copy(ei_hbm.at[0, pl.ds(base, IDXCHUNK)], srcbuf)
        pltpu.sync_copy(ei_hbm.at[1, pl.ds(base, IDXCHUNK)], dstbuf)
        pltpu.async_copy(table_hbm.at[srcbuf.at[0]], rows0, sem0)

        @pl.loop(0, IDXCHUNK, step=2)
        def _edge(j):
            pltpu.async_copy(table_hbm.at[srcbuf.at[j + 1]], rows1, sem1)
            pltpu.make_async_copy(table_hbm.at[srcbuf.at[j]], rows0,
                                  sem0).wait()
            pltpu.sync_copy(rows0, acc.at[dstbuf.at[j]], add=True)

            @pl.when(j + 2 < IDXCHUNK)
            def _():
                pltpu.async_copy(table_hbm.at[srcbuf.at[j + 2]], rows0, sem0)

            pltpu.make_async_copy(table_hbm.at[srcbuf.at[j + 1]], rows1,
                                  sem1).wait()
            pltpu.sync_copy(rows1, acc.at[dstbuf.at[j + 1]], add=True)

    plsc.subcore_barrier()

    @pl.when(s < NTILES - 1)
    def _():
        pltpu.sync_copy(acc.at[pl.ds(RPT * s, RPT)],
                        part_hbm.at[c, pl.ds(RPT * s, RPT)])

    @pl.when(s == NTILES - 1)
    def _():
        tail = N - RPT * (NTILES - 1)
        pltpu.sync_copy(acc.at[pl.ds(RPT * (NTILES - 1), tail)],
                        part_hbm.at[c, pl.ds(RPT * (NTILES - 1), tail)])


_sc_agg = functools.partial(
    pl.kernel,
    out_type=jax.ShapeDtypeStruct((NSC, N, D), jnp.float32),
    mesh=plsc.VectorSubcoreMesh(core_axis_name="c", subcore_axis_name="s"),
    scratch_types=[
        pltpu.VMEM_SHARED((ACC_ROWS, D), jnp.float32),
        pltpu.VMEM((IDXCHUNK, STREAM), jnp.int32),
        pltpu.VMEM((IDXCHUNK, STREAM), jnp.int32),
        pltpu.VMEM((2, STREAM, D), jnp.float32),
        pltpu.SemaphoreType.DMA,
        pltpu.SemaphoreType.DMA,
    ],
)(_sc_agg_body)


ROWB = 1000


def _tc_hidden_body(p01, xp, dinv, w1, b1, o):
    p = p01[...]
    t = dinv[...] * (p[0] + p[1] + xp[...])
    h = jnp.dot(t, w1[...], preferred_element_type=jnp.float32) + b1[...]
    o[...] = dinv[...] * jnp.maximum(h, 0.0)


def _tc_hidden(p01, xp, dinv, w1, b1):
    grid = (N // ROWB,)
    row_spec = pl.BlockSpec((ROWB, D), lambda i: (i, 0))
    return pl.pallas_call(
        _tc_hidden_body,
        grid=grid,
        in_specs=[pl.BlockSpec((2, ROWB, D), lambda i: (0, i, 0)),
                  row_spec,
                  pl.BlockSpec((ROWB, 1), lambda i: (i, 0)),
                  pl.BlockSpec((D, D), lambda i: (0, 0)),
                  pl.BlockSpec((1, D), lambda i: (0, 0))],
        out_specs=row_spec,
        out_shape=jax.ShapeDtypeStruct((N, D), jnp.float32),
    )(p01, xp, dinv, w1, b1)


def _tc_heads_body(q01, hp, dinv, wmu, bmu, wlv, blv, eps, z, mu, lv):
    q = q01[...]
    a = dinv[...] * (q[0] + q[1] + hp[...])
    m = jnp.dot(a, wmu[...], preferred_element_type=jnp.float32) + bmu[...]
    v = jnp.dot(a, wlv[...], preferred_element_type=jnp.float32) + blv[...]
    mu[...] = m
    lv[...] = v
    z[...] = m + jnp.exp(0.5 * v) * eps[...]


def _tc_heads(q01, hp, dinv, wmu, bmu, wlv, blv, eps):
    grid = (N // ROWB,)
    row_spec = pl.BlockSpec((ROWB, D), lambda i: (i, 0))
    z_spec = pl.BlockSpec((ROWB, DZ), lambda i: (i, 0))
    w_spec = pl.BlockSpec((D, DZ), lambda i: (0, 0))
    b_spec = pl.BlockSpec((1, DZ), lambda i: (0, 0))
    zshape = jax.ShapeDtypeStruct((N, DZ), jnp.float32)
    return pl.pallas_call(
        _tc_heads_body,
        grid=grid,
        in_specs=[pl.BlockSpec((2, ROWB, D), lambda i: (0, i, 0)),
                  row_spec,
                  pl.BlockSpec((ROWB, 1), lambda i: (i, 0)),
                  w_spec, b_spec, w_spec, b_spec, z_spec],
        out_specs=(z_spec, z_spec, z_spec),
        out_shape=(zshape, zshape, zshape),
    )(q01, hp, dinv, wmu, bmu, wlv, blv, eps)


def kernel(x, edge_index, W1, b1, Wmu, bmu, Wlv, blv):
    pad = EPAD - E
    src_pad = jnp.arange(pad, dtype=jnp.int32)
    dst_pad = N + jnp.broadcast_to(
        jnp.arange(ACC_ROWS - N, dtype=jnp.int32),
        (pad // (ACC_ROWS - N), ACC_ROWS - N)).reshape(-1)
    ei3 = jnp.concatenate(
        [edge_index.astype(jnp.int32),
         jnp.stack([src_pad, dst_pad])], axis=1).reshape(2, NSTREAMS, STREAM)

    degp = _sc_deg(ei3)
    dinv, xp = _sc_prep(x, degp)
    dinv2d = dinv.reshape(N, 1)
    zrows = jnp.zeros((RPT, D), jnp.float32)

    p = _sc_agg(xp, ei3, zrows)
    hp = _tc_hidden(p, xp, dinv2d, W1, b1.reshape(1, D))

    q = _sc_agg(hp, ei3, zrows)
    eps = jax.random.normal(jax.random.key(42), (N, DZ), jnp.float32)
    z, mu, lv = _tc_heads(q, hp, dinv2d,
                          Wmu, bmu.reshape(1, DZ),
                          Wlv, blv.reshape(1, DZ), eps)
    return z, mu, lv


# speedup vs baseline: 1.0672x; 1.0163x over previous
import functools

import jax
import jax.numpy as jnp
from jax import lax
from jax.experimental import pallas as pl
from jax.experimental.pallas import tpu as pltpu
from jax.experimental.pallas import tpu_sc as plsc

N = 10000
E = 320000
D = 128
DZ = 64

NSC = 2
NTILES = 16
NW = NSC * NTILES

STREAM = 128
NSTREAMS = 2560
EPAD = NSTREAMS * STREAM
SPT_AGG = NSTREAMS // NW
SPT_DEG = NSTREAMS // NTILES
ACC_ROWS = 10240
RPT = ACC_ROWS // NTILES
GARBAGE = N
IDXCHUNK = 40


def _rsqrt16(d):
    i = lax.bitcast_convert_type(d, jnp.int32)
    i = jnp.int32(0x5F3759DF) - lax.shift_right_logical(i, jnp.int32(1))
    y = lax.bitcast_convert_type(i, jnp.float32)
    for _ in range(3):
        y = y * (1.5 - 0.5 * d * y * y)
    return y


def _sc_deg_body(ei_hbm, degp_hbm, deg_acc, zb, ones_v, dstbuf, dsem):
    c = lax.axis_index("c")
    s = lax.axis_index("s")
    w = c * NTILES + s

    for k in range(RPT // 16):
        zb[pl.ds(16 * k, 16)] = jnp.zeros((16,), jnp.float32)
    pltpu.sync_copy(zb, deg_acc.at[pl.ds(RPT * s, RPT)])
    for k in range(STREAM // 16):
        ones_v[pl.ds(16 * k, 16)] = jnp.ones((16,), jnp.float32)
    plsc.subcore_barrier()

    pltpu.sync_copy(ei_hbm.at[1, pl.ds(SPT_AGG * w, SPT_AGG)], dstbuf)
    WIN = 32

    @pl.loop(0, SPT_AGG)
    def _deg(j):
        pltpu.async_copy(ones_v, deg_acc.at[dstbuf.at[j]], dsem, add=True)

        @pl.when(j >= WIN)
        def _():
            pltpu.make_async_copy(ones_v, deg_acc.at[dstbuf.at[0]],
                                  dsem).wait()

    @pl.loop(0, WIN)
    def _drain(j):
        pltpu.make_async_copy(ones_v, deg_acc.at[dstbuf.at[0]], dsem).wait()

    plsc.subcore_barrier()
    pltpu.sync_copy(deg_acc.at[pl.ds(RPT * s, RPT)],
                    degp_hbm.at[pl.ds(c * ACC_ROWS + RPT * s, RPT)])


_sc_deg = functools.partial(
    pl.kernel,
    out_type=jax.ShapeDtypeStruct((NSC * ACC_ROWS,), jnp.float32),
    mesh=plsc.VectorSubcoreMesh(core_axis_name="c", subcore_axis_name="s"),
    scratch_types=[
        pltpu.VMEM_SHARED((ACC_ROWS,), jnp.float32),
        pltpu.VMEM((RPT,), jnp.float32),
        pltpu.VMEM((STREAM,), jnp.float32),
        pltpu.VMEM((SPT_AGG, STREAM), jnp.int32),
        pltpu.SemaphoreType.DMA,
    ],
)(_sc_deg_body)


RPW = ACC_ROWS // NW


def _sc_prep_body(x_hbm, degp_hbm, dinv_hbm, xp_hbm,
                  degbuf, deg2buf, dinvbuf, xbuf):
    c = lax.axis_index("c")
    s = lax.axis_index("s")
    w = c * NTILES + s

    pltpu.sync_copy(degp_hbm.at[pl.ds(RPW * w, RPW)], degbuf)
    pltpu.sync_copy(degp_hbm.at[pl.ds(ACC_ROWS + RPW * w, RPW)], deg2buf)
    for k in range(RPW // 16):
        dvec = degbuf[pl.ds(16 * k, 16)] + deg2buf[pl.ds(16 * k, 16)] + 1.0
        dinvbuf[pl.ds(16 * k, 16)] = _rsqrt16(dvec)

    nrows = jnp.minimum(RPW, jnp.maximum(N - RPW * w, 0))

    @pl.when(nrows == RPW)
    def _():
        pltpu.sync_copy(dinvbuf.at[pl.ds(0, RPW)],
                        dinv_hbm.at[pl.ds(RPW * w, RPW)])

    @pl.when(jnp.logical_and(nrows > 0, nrows < RPW))
    def _():
        tail = N - RPW * (NW - 1)
        pltpu.sync_copy(dinvbuf.at[pl.ds(0, tail)],
                        dinv_hbm.at[pl.ds(RPW * (NW - 1), tail)])

    nchunks = nrows // 16

    @pl.loop(0, RPW // 16)
    def _chunk(i):
        @pl.when(i < nchunks)
        def _():
            row0 = RPW * w + 16 * i
            pltpu.sync_copy(x_hbm.at[pl.ds(row0, 16)], xbuf)
            v16 = dinvbuf[pl.ds(16 * i, 16)]
            for r in range(16):
                dv = jnp.full((16,), v16[r], jnp.float32)
                for jj in range(D // 16):
                    sl = pl.ds(16 * jj, 16)
                    xbuf[r, sl] = xbuf[r, sl] * dv
            pltpu.sync_copy(xbuf, xp_hbm.at[pl.ds(row0, 16)])


_sc_prep = functools.partial(
    pl.kernel,
    out_type=(jax.ShapeDtypeStruct((N,), jnp.float32),
              jax.ShapeDtypeStruct((N, D), jnp.float32)),
    mesh=plsc.VectorSubcoreMesh(core_axis_name="c", subcore_axis_name="s"),
    scratch_types=[
        pltpu.VMEM((RPW,), jnp.float32),
        pltpu.VMEM((RPW,), jnp.float32),
        pltpu.VMEM((RPW + 16,), jnp.float32),
        pltpu.VMEM((16, D), jnp.float32),
    ],
)(_sc_prep_body)


def _sc_agg_body(table_hbm, ei_hbm, zrows_hbm, part_hbm,
                 acc, srcbuf, dstbuf, rows, sem0, sem1):
    c = lax.axis_index("c")
    s = lax.axis_index("s")
    w = c * NTILES + s

    pltpu.sync_copy(zrows_hbm, acc.at[pl.ds(RPT * s, RPT)])
    plsc.subcore_barrier()

    rows0 = rows.at[0]
    rows1 = rows.at[1]
    for k in range(SPT_AGG // IDXCHUNK):
        base = SPT_AGG * w + IDXCHUNK * k
        pltpu.sync_copy(ei_hbm.at[0, pl.ds(base, IDXCHUNK)], srcbuf)
        pltpu.sync_copy(ei_hbm.at[1, pl.ds(base, IDXCHUNK)], dstbuf)
        pltpu.async_copy(table_hbm.at[srcbuf.at[0]], rows0, sem0)

        @pl.loop(0, IDXCHUNK, step=2)
        def _edge(j):
            pltpu.async_copy(table_hbm.at[srcbuf.at[j + 1]], rows1, sem1)
            pltpu.make_async_copy(table_hbm.at[srcbuf.at[j]], rows0,
                                  sem0).wait()
            pltpu.sync_copy(rows0, acc.at[dstbuf.at[j]], add=True)

            @pl.when(j + 2 < IDXCHUNK)
            def _():
                pltpu.async_copy(table_hbm.at[srcbuf.at[j + 2]], rows0, sem0)

            pltpu.make_async_copy(table_hbm.at[srcbuf.at[j + 1]], rows1,
                                  sem1).wait()
            pltpu.sync_copy(rows1, acc.at[dstbuf.at[j + 1]], add=True)

    plsc.subcore_barrier()

    @pl.when(s < NTILES - 1)
    def _():
        pltpu.sync_copy(acc.at[pl.ds(RPT * s, RPT)],
                        part_hbm.at[c, pl.ds(RPT * s, RPT)])

    @pl.when(s == NTILES - 1)
    def _():
        tail = N - RPT * (NTILES - 1)
        pltpu.sync_copy(acc.at[pl.ds(RPT * (NTILES - 1), tail)],
                        part_hbm.at[c, pl.ds(RPT * (NTILES - 1), tail)])


_sc_agg = functools.partial(
    pl.kernel,
    out_type=jax.ShapeDtypeStruct((NSC, N, D), jnp.float32),
    mesh=plsc.VectorSubcoreMesh(core_axis_name="c", subcore_axis_name="s"),
    scratch_types=[
        pltpu.VMEM_SHARED((ACC_ROWS, D), jnp.float32),
        pltpu.VMEM((IDXCHUNK, STREAM), jnp.int32),
        pltpu.VMEM((IDXCHUNK, STREAM), jnp.int32),
        pltpu.VMEM((2, STREAM, D), jnp.float32),
        pltpu.SemaphoreType.DMA,
        pltpu.SemaphoreType.DMA,
    ],
)(_sc_agg_body)


ROWB = 2000


def _tc_hidden_body(p01, xp, dinv, w1, b1, o):
    p = p01[...]
    t = dinv[...] * (p[0] + p[1] + xp[...])
    h = jnp.dot(t, w1[...], preferred_element_type=jnp.float32) + b1[...]
    o[...] = dinv[...] * jnp.maximum(h, 0.0)


def _tc_hidden(p01, xp, dinv, w1, b1):
    grid = (N // ROWB,)
    row_spec = pl.BlockSpec((ROWB, D), lambda i: (i, 0))
    return pl.pallas_call(
        _tc_hidden_body,
        grid=grid,
        in_specs=[pl.BlockSpec((2, ROWB, D), lambda i: (0, i, 0)),
                  row_spec,
                  pl.BlockSpec((ROWB, 1), lambda i: (i, 0)),
                  pl.BlockSpec((D, D), lambda i: (0, 0)),
                  pl.BlockSpec((1, D), lambda i: (0, 0))],
        out_specs=row_spec,
        out_shape=jax.ShapeDtypeStruct((N, D), jnp.float32),
    )(p01, xp, dinv, w1, b1)


def _tc_heads_body(q01, hp, dinv, wmu, bmu, wlv, blv, eps, z, mu, lv):
    q = q01[...]
    a = dinv[...] * (q[0] + q[1] + hp[...])
    m = jnp.dot(a, wmu[...], preferred_element_type=jnp.float32) + bmu[...]
    v = jnp.dot(a, wlv[...], preferred_element_type=jnp.float32) + blv[...]
    mu[...] = m
    lv[...] = v
    z[...] = m + jnp.exp(0.5 * v) * eps[...]


def _tc_heads(q01, hp, dinv, wmu, bmu, wlv, blv, eps):
    grid = (N // ROWB,)
    row_spec = pl.BlockSpec((ROWB, D), lambda i: (i, 0))
    z_spec = pl.BlockSpec((ROWB, DZ), lambda i: (i, 0))
    w_spec = pl.BlockSpec((D, DZ), lambda i: (0, 0))
    b_spec = pl.BlockSpec((1, DZ), lambda i: (0, 0))
    zshape = jax.ShapeDtypeStruct((N, DZ), jnp.float32)
    return pl.pallas_call(
        _tc_heads_body,
        grid=grid,
        in_specs=[pl.BlockSpec((2, ROWB, D), lambda i: (0, i, 0)),
                  row_spec,
                  pl.BlockSpec((ROWB, 1), lambda i: (i, 0)),
                  w_spec, b_spec, w_spec, b_spec, z_spec],
        out_specs=(z_spec, z_spec, z_spec),
        out_shape=(zshape, zshape, zshape),
    )(q01, hp, dinv, wmu, bmu, wlv, blv, eps)


def kernel(x, edge_index, W1, b1, Wmu, bmu, Wlv, blv):
    pad = EPAD - E
    src_pad = jnp.arange(pad, dtype=jnp.int32)
    dst_pad = N + jnp.broadcast_to(
        jnp.arange(ACC_ROWS - N, dtype=jnp.int32),
        (pad // (ACC_ROWS - N), ACC_ROWS - N)).reshape(-1)
    ei3 = jnp.concatenate(
        [edge_index.astype(jnp.int32),
         jnp.stack([src_pad, dst_pad])], axis=1).reshape(2, NSTREAMS, STREAM)

    degp = _sc_deg(ei3)
    dinv, xp = _sc_prep(x, degp)
    dinv2d = dinv.reshape(N, 1)
    zrows = jnp.zeros((RPT, D), jnp.float32)

    p = _sc_agg(xp, ei3, zrows)
    hp = _tc_hidden(p, xp, dinv2d, W1, b1.reshape(1, D))

    q = _sc_agg(hp, ei3, zrows)
    eps = jax.random.normal(jax.random.key(42), (N, DZ), jnp.float32)
    z, mu, lv = _tc_heads(q, hp, dinv2d,
                          Wmu, bmu.reshape(1, DZ),
                          Wlv, blv.reshape(1, DZ), eps)
    return z, mu, lv


# speedup vs baseline: 1.1251x; 1.0542x over previous
import functools

import jax
import jax.numpy as jnp
from jax import lax
from jax.experimental import pallas as pl
from jax.experimental.pallas import tpu as pltpu
from jax.experimental.pallas import tpu_sc as plsc

N = 10000
E = 320000
D = 128
DZ = 64

NSC = 2
NTILES = 16
NW = NSC * NTILES

STREAM = 128
NSTREAMS = 2560
EPAD = NSTREAMS * STREAM
SPT_AGG = NSTREAMS // NW
SPT_DEG = NSTREAMS // NTILES
ACC_ROWS = 10240
RPT = ACC_ROWS // NTILES
GARBAGE = N
IDXCHUNK = 40


def _rsqrt16(d):
    i = lax.bitcast_convert_type(d, jnp.int32)
    i = jnp.int32(0x5F3759DF) - lax.shift_right_logical(i, jnp.int32(1))
    y = lax.bitcast_convert_type(i, jnp.float32)
    for _ in range(3):
        y = y * (1.5 - 0.5 * d * y * y)
    return y


def _sc_deg_body(ei_hbm, degp_hbm, deg_acc, zb, ones_v, dstbuf, dsem):
    c = lax.axis_index("c")
    s = lax.axis_index("s")
    w = c * NTILES + s

    for k in range(RPT // 16):
        zb[pl.ds(16 * k, 16)] = jnp.zeros((16,), jnp.float32)
    pltpu.sync_copy(zb, deg_acc.at[pl.ds(RPT * s, RPT)])
    for k in range(STREAM // 16):
        ones_v[pl.ds(16 * k, 16)] = jnp.ones((16,), jnp.float32)
    plsc.subcore_barrier()

    pltpu.sync_copy(ei_hbm.at[1, pl.ds(SPT_AGG * w, SPT_AGG)], dstbuf)
    WIN = 32

    @pl.loop(0, SPT_AGG)
    def _deg(j):
        pltpu.async_copy(ones_v, deg_acc.at[dstbuf.at[j]], dsem, add=True)

        @pl.when(j >= WIN)
        def _():
            pltpu.make_async_copy(ones_v, deg_acc.at[dstbuf.at[0]],
                                  dsem).wait()

    @pl.loop(0, WIN)
    def _drain(j):
        pltpu.make_async_copy(ones_v, deg_acc.at[dstbuf.at[0]], dsem).wait()

    plsc.subcore_barrier()
    pltpu.sync_copy(deg_acc.at[pl.ds(RPT * s, RPT)],
                    degp_hbm.at[pl.ds(c * ACC_ROWS + RPT * s, RPT)])


_sc_deg = functools.partial(
    pl.kernel,
    out_type=jax.ShapeDtypeStruct((NSC * ACC_ROWS,), jnp.float32),
    mesh=plsc.VectorSubcoreMesh(core_axis_name="c", subcore_axis_name="s"),
    scratch_types=[
        pltpu.VMEM_SHARED((ACC_ROWS,), jnp.float32),
        pltpu.VMEM((RPT,), jnp.float32),
        pltpu.VMEM((STREAM,), jnp.float32),
        pltpu.VMEM((SPT_AGG, STREAM), jnp.int32),
        pltpu.SemaphoreType.DMA,
    ],
)(_sc_deg_body)


RPW = ACC_ROWS // NW


def _sc_prep_body(x_hbm, degp_hbm, dinv_hbm, xp_hbm,
                  degbuf, deg2buf, dinvbuf, xbuf):
    c = lax.axis_index("c")
    s = lax.axis_index("s")
    w = c * NTILES + s

    pltpu.sync_copy(degp_hbm.at[pl.ds(RPW * w, RPW)], degbuf)
    pltpu.sync_copy(degp_hbm.at[pl.ds(ACC_ROWS + RPW * w, RPW)], deg2buf)
    for k in range(RPW // 16):
        dvec = degbuf[pl.ds(16 * k, 16)] + deg2buf[pl.ds(16 * k, 16)] + 1.0
        dinvbuf[pl.ds(16 * k, 16)] = _rsqrt16(dvec)

    nrows = jnp.minimum(RPW, jnp.maximum(N - RPW * w, 0))

    @pl.when(nrows == RPW)
    def _():
        pltpu.sync_copy(dinvbuf.at[pl.ds(0, RPW)],
                        dinv_hbm.at[pl.ds(RPW * w, RPW)])

    @pl.when(jnp.logical_and(nrows > 0, nrows < RPW))
    def _():
        tail = N - RPW * (NW - 1)
        pltpu.sync_copy(dinvbuf.at[pl.ds(0, tail)],
                        dinv_hbm.at[pl.ds(RPW * (NW - 1), tail)])

    nchunks = nrows // 16

    @pl.loop(0, RPW // 16)
    def _chunk(i):
        @pl.when(i < nchunks)
        def _():
            row0 = RPW * w + 16 * i
            pltpu.sync_copy(x_hbm.at[pl.ds(row0, 16)], xbuf)
            v16 = dinvbuf[pl.ds(16 * i, 16)]
            for r in range(16):
                dv = jnp.full((16,), v16[r], jnp.float32)
                for jj in range(D // 16):
                    sl = pl.ds(16 * jj, 16)
                    xbuf[r, sl] = xbuf[r, sl] * dv
            pltpu.sync_copy(xbuf, xp_hbm.at[pl.ds(row0, 16)])


_sc_prep = functools.partial(
    pl.kernel,
    out_type=(jax.ShapeDtypeStruct((N,), jnp.float32),
              jax.ShapeDtypeStruct((N, D), jnp.float32)),
    mesh=plsc.VectorSubcoreMesh(core_axis_name="c", subcore_axis_name="s"),
    scratch_types=[
        pltpu.VMEM((RPW,), jnp.float32),
        pltpu.VMEM((RPW,), jnp.float32),
        pltpu.VMEM((RPW + 16,), jnp.float32),
        pltpu.VMEM((16, D), jnp.float32),
    ],
)(_sc_prep_body)


def _sc_agg_body(table_hbm, ei_hbm, zrows_hbm, part_hbm,
                 acc, srcbuf, dstbuf, rows, sem0, sem1):
    c = lax.axis_index("c")
    s = lax.axis_index("s")
    w = c * NTILES + s

    pltpu.sync_copy(zrows_hbm, acc.at[pl.ds(RPT * s, RPT)])
    plsc.subcore_barrier()

    rows0 = rows.at[0]
    rows1 = rows.at[1]
    for k in range(SPT_AGG // IDXCHUNK):
        base = SPT_AGG * w + IDXCHUNK * k
        pltpu.sync_copy(ei_hbm.at[0, pl.ds(base, IDXCHUNK)], srcbuf)
        pltpu.sync_copy(ei_hbm.at[1, pl.ds(base, IDXCHUNK)], dstbuf)
        pltpu.async_copy(table_hbm.at[srcbuf.at[0]], rows0, sem0)

        @pl.loop(0, IDXCHUNK, step=2)
        def _edge(j):
            pltpu.async_copy(table_hbm.at[srcbuf.at[j + 1]], rows1, sem1)
            pltpu.make_async_copy(table_hbm.at[srcbuf.at[j]], rows0,
                                  sem0).wait()
            pltpu.sync_copy(rows0, acc.at[dstbuf.at[j]], add=True)

            @pl.when(j + 2 < IDXCHUNK)
            def _():
                pltpu.async_copy(table_hbm.at[srcbuf.at[j + 2]], rows0, sem0)

            pltpu.make_async_copy(table_hbm.at[srcbuf.at[j + 1]], rows1,
                                  sem1).wait()
            pltpu.sync_copy(rows1, acc.at[dstbuf.at[j + 1]], add=True)

    plsc.subcore_barrier()

    @pl.when(s < NTILES - 1)
    def _():
        pltpu.sync_copy(acc.at[pl.ds(RPT * s, RPT)],
                        part_hbm.at[c, pl.ds(RPT * s, RPT)])

    @pl.when(s == NTILES - 1)
    def _():
        tail = N - RPT * (NTILES - 1)
        pltpu.sync_copy(acc.at[pl.ds(RPT * (NTILES - 1), tail)],
                        part_hbm.at[c, pl.ds(RPT * (NTILES - 1), tail)])


_sc_agg = functools.partial(
    pl.kernel,
    out_type=jax.ShapeDtypeStruct((NSC, N, D), jnp.float32),
    mesh=plsc.VectorSubcoreMesh(core_axis_name="c", subcore_axis_name="s"),
    scratch_types=[
        pltpu.VMEM_SHARED((ACC_ROWS, D), jnp.float32),
        pltpu.VMEM((IDXCHUNK, STREAM), jnp.int32),
        pltpu.VMEM((IDXCHUNK, STREAM), jnp.int32),
        pltpu.VMEM((2, STREAM, D), jnp.float32),
        pltpu.SemaphoreType.DMA,
        pltpu.SemaphoreType.DMA,
    ],
)(_sc_agg_body)


ROWB = 2048
TCGRID = (N + ROWB - 1) // ROWB


def _tc_hidden_body(p01, xp, dinv, w1, b1, o):
    p = p01[...]
    t = dinv[...] * (p[0] + p[1] + xp[...])
    h = jnp.dot(t, w1[...], preferred_element_type=jnp.float32) + b1[...]
    o[...] = dinv[...] * jnp.maximum(h, 0.0)


def _tc_hidden(p01, xp, dinv, w1, b1):
    grid = (TCGRID,)
    row_spec = pl.BlockSpec((ROWB, D), lambda i: (i, 0))
    return pl.pallas_call(
        _tc_hidden_body,
        grid=grid,
        in_specs=[pl.BlockSpec((2, ROWB, D), lambda i: (0, i, 0)),
                  row_spec,
                  pl.BlockSpec((ROWB, 1), lambda i: (i, 0)),
                  pl.BlockSpec((D, D), lambda i: (0, 0)),
                  pl.BlockSpec((1, D), lambda i: (0, 0))],
        out_specs=row_spec,
        out_shape=jax.ShapeDtypeStruct((N, D), jnp.float32),
    )(p01, xp, dinv, w1, b1)


def _tc_heads_body(q01, hp, dinv, wmut, bmu, wlvt, blv, epst, z, mu, lv):
    q = q01[...]
    a = dinv[...] * (q[0] + q[1] + hp[...])
    nt = (((1,), (1,)), ((), ()))
    m = lax.dot_general(wmut[...], a, nt,
                        preferred_element_type=jnp.float32) + bmu[...]
    v = lax.dot_general(wlvt[...], a, nt,
                        preferred_element_type=jnp.float32) + blv[...]
    mu[...] = m
    lv[...] = v
    z[...] = m + jnp.exp(0.5 * v) * epst[...]


def _tc_heads(q01, hp, dinv, wmut, bmu, wlvt, blv, epst):
    grid = (TCGRID,)
    row_spec = pl.BlockSpec((ROWB, D), lambda i: (i, 0))
    z_spec = pl.BlockSpec((DZ, ROWB), lambda i: (0, i))
    w_spec = pl.BlockSpec((DZ, D), lambda i: (0, 0))
    b_spec = pl.BlockSpec((DZ, 1), lambda i: (0, 0))
    zshape = jax.ShapeDtypeStruct((DZ, N), jnp.float32)
    return pl.pallas_call(
        _tc_heads_body,
        grid=grid,
        in_specs=[pl.BlockSpec((2, ROWB, D), lambda i: (0, i, 0)),
                  row_spec,
                  pl.BlockSpec((ROWB, 1), lambda i: (i, 0)),
                  w_spec, b_spec, w_spec, b_spec, z_spec],
        out_specs=(z_spec, z_spec, z_spec),
        out_shape=(zshape, zshape, zshape),
    )(q01, hp, dinv, wmut, bmu, wlvt, blv, epst)


def kernel(x, edge_index, W1, b1, Wmu, bmu, Wlv, blv):
    pad = EPAD - E
    src_pad = jnp.arange(pad, dtype=jnp.int32)
    dst_pad = N + jnp.broadcast_to(
        jnp.arange(ACC_ROWS - N, dtype=jnp.int32),
        (pad // (ACC_ROWS - N), ACC_ROWS - N)).reshape(-1)
    ei3 = jnp.concatenate(
        [edge_index.astype(jnp.int32),
         jnp.stack([src_pad, dst_pad])], axis=1).reshape(2, NSTREAMS, STREAM)

    degp = _sc_deg(ei3)
    dinv, xp = _sc_prep(x, degp)
    dinv2d = dinv.reshape(N, 1)
    zrows = jnp.zeros((RPT, D), jnp.float32)

    p = _sc_agg(xp, ei3, zrows)
    hp = _tc_hidden(p, xp, dinv2d, W1, b1.reshape(1, D))

    q = _sc_agg(hp, ei3, zrows)
    epst = jax.random.normal(jax.random.key(42), (N, DZ), jnp.float32).T
    zt, mut, lvt = _tc_heads(q, hp, dinv2d,
                             Wmu.T, bmu.reshape(DZ, 1),
                             Wlv.T, blv.reshape(DZ, 1), epst)
    return zt.T, mut.T, lvt.T


# speedup vs baseline: 1.1576x; 1.0289x over previous
import functools

import jax
import jax.numpy as jnp
from jax import lax
from jax.experimental import pallas as pl
from jax.experimental.pallas import tpu as pltpu
from jax.experimental.pallas import tpu_sc as plsc

N = 10000
E = 320000
D = 128
DZ = 64

NSC = 2
NTILES = 16
NW = NSC * NTILES

STREAM = 128
NSTREAMS = 2560
EPAD = NSTREAMS * STREAM
SPT_AGG = NSTREAMS // NW
SPT_DEG = NSTREAMS // NTILES
ACC_ROWS = 10240
RPT = ACC_ROWS // NTILES
GARBAGE = N
IDXCHUNK = 40


def _rsqrt16(d):
    i = lax.bitcast_convert_type(d, jnp.int32)
    i = jnp.int32(0x5F3759DF) - lax.shift_right_logical(i, jnp.int32(1))
    y = lax.bitcast_convert_type(i, jnp.float32)
    for _ in range(3):
        y = y * (1.5 - 0.5 * d * y * y)
    return y


def _sc_deg_body(ei_hbm, degp_hbm, deg_acc, zb, ones_v, dstbuf, dsem):
    c = lax.axis_index("c")
    s = lax.axis_index("s")
    w = c * NTILES + s

    for k in range(RPT // 16):
        zb[pl.ds(16 * k, 16)] = jnp.zeros((16,), jnp.float32)
    pltpu.sync_copy(zb, deg_acc.at[pl.ds(RPT * s, RPT)])
    for k in range(STREAM // 16):
        ones_v[pl.ds(16 * k, 16)] = jnp.ones((16,), jnp.float32)
    plsc.subcore_barrier()

    pltpu.sync_copy(ei_hbm.at[1, pl.ds(SPT_AGG * w, SPT_AGG)], dstbuf)
    WIN = 32

    @pl.loop(0, SPT_AGG)
    def _deg(j):
        pltpu.async_copy(ones_v, deg_acc.at[dstbuf.at[j]], dsem, add=True)

        @pl.when(j >= WIN)
        def _():
            pltpu.make_async_copy(ones_v, deg_acc.at[dstbuf.at[0]],
                                  dsem).wait()

    @pl.loop(0, WIN)
    def _drain(j):
        pltpu.make_async_copy(ones_v, deg_acc.at[dstbuf.at[0]], dsem).wait()

    plsc.subcore_barrier()
    pltpu.sync_copy(deg_acc.at[pl.ds(RPT * s, RPT)],
                    degp_hbm.at[pl.ds(c * ACC_ROWS + RPT * s, RPT)])


_sc_deg = functools.partial(
    pl.kernel,
    out_type=jax.ShapeDtypeStruct((NSC * ACC_ROWS,), jnp.float32),
    mesh=plsc.VectorSubcoreMesh(core_axis_name="c", subcore_axis_name="s"),
    scratch_types=[
        pltpu.VMEM_SHARED((ACC_ROWS,), jnp.float32),
        pltpu.VMEM((RPT,), jnp.float32),
        pltpu.VMEM((STREAM,), jnp.float32),
        pltpu.VMEM((SPT_AGG, STREAM), jnp.int32),
        pltpu.SemaphoreType.DMA,
    ],
)(_sc_deg_body)


RPW = ACC_ROWS // NW


def _sc_prep_body(x_hbm, degp_hbm, dinv_hbm, xp_hbm,
                  degbuf, deg2buf, dinvbuf, xbuf, xsem0, xsem1):
    c = lax.axis_index("c")
    s = lax.axis_index("s")
    w = c * NTILES + s

    pltpu.sync_copy(degp_hbm.at[pl.ds(RPW * w, RPW)], degbuf)
    pltpu.sync_copy(degp_hbm.at[pl.ds(ACC_ROWS + RPW * w, RPW)], deg2buf)
    for k in range(RPW // 16):
        dvec = degbuf[pl.ds(16 * k, 16)] + deg2buf[pl.ds(16 * k, 16)] + 1.0
        dinvbuf[pl.ds(16 * k, 16)] = _rsqrt16(dvec)

    nrows = jnp.minimum(RPW, jnp.maximum(N - RPW * w, 0))

    @pl.when(nrows == RPW)
    def _():
        pltpu.sync_copy(dinvbuf.at[pl.ds(0, RPW)],
                        dinv_hbm.at[pl.ds(RPW * w, RPW)])

    @pl.when(jnp.logical_and(nrows > 0, nrows < RPW))
    def _():
        tail = N - RPW * (NW - 1)
        pltpu.sync_copy(dinvbuf.at[pl.ds(0, tail)],
                        dinv_hbm.at[pl.ds(RPW * (NW - 1), tail)])

    nchunks = nrows // 16
    xb0 = xbuf.at[0]
    xb1 = xbuf.at[1]

    def _scale_store(xb, i):
        v16 = dinvbuf[pl.ds(16 * i, 16)]
        for r in range(16):
            dv = jnp.full((16,), v16[r], jnp.float32)
            for jj in range(D // 16):
                sl = pl.ds(16 * jj, 16)
                xb[r, sl] = xb[r, sl] * dv
        pltpu.sync_copy(xb, xp_hbm.at[pl.ds(RPW * w + 16 * i, 16)])

    @pl.when(nchunks > 0)
    def _():
        pltpu.async_copy(x_hbm.at[pl.ds(RPW * w, 16)], xb0, xsem0)

    @pl.loop(0, RPW // 16, step=2)
    def _chunk(i):
        @pl.when(i < nchunks)
        def _():
            @pl.when(i + 1 < nchunks)
            def _():
                pltpu.async_copy(
                    x_hbm.at[pl.ds(RPW * w + 16 * (i + 1), 16)], xb1, xsem1)

            pltpu.make_async_copy(x_hbm.at[pl.ds(0, 16)], xb0, xsem0).wait()
            _scale_store(xb0, i)

            @pl.when(i + 2 < nchunks)
            def _():
                pltpu.async_copy(
                    x_hbm.at[pl.ds(RPW * w + 16 * (i + 2), 16)], xb0, xsem0)

            @pl.when(i + 1 < nchunks)
            def _():
                pltpu.make_async_copy(x_hbm.at[pl.ds(0, 16)], xb1,
                                      xsem1).wait()
                _scale_store(xb1, i + 1)


_sc_prep = functools.partial(
    pl.kernel,
    out_type=(jax.ShapeDtypeStruct((N,), jnp.float32),
              jax.ShapeDtypeStruct((N, D), jnp.float32)),
    mesh=plsc.VectorSubcoreMesh(core_axis_name="c", subcore_axis_name="s"),
    scratch_types=[
        pltpu.VMEM((RPW,), jnp.float32),
        pltpu.VMEM((RPW,), jnp.float32),
        pltpu.VMEM((RPW + 16,), jnp.float32),
        pltpu.VMEM((2, 16, D), jnp.float32),
        pltpu.SemaphoreType.DMA,
        pltpu.SemaphoreType.DMA,
    ],
)(_sc_prep_body)


def _sc_agg_body(table_hbm, ei_hbm, zrows_hbm, part_hbm,
                 acc, srcbuf, dstbuf, rows, sem0, sem1):
    c = lax.axis_index("c")
    s = lax.axis_index("s")
    w = c * NTILES + s

    pltpu.sync_copy(zrows_hbm, acc.at[pl.ds(RPT * s, RPT)])
    plsc.subcore_barrier()

    rows0 = rows.at[0]
    rows1 = rows.at[1]
    for k in range(SPT_AGG // IDXCHUNK):
        base = SPT_AGG * w + IDXCHUNK * k
        pltpu.sync_copy(ei_hbm.at[0, pl.ds(base, IDXCHUNK)], srcbuf)
        pltpu.sync_copy(ei_hbm.at[1, pl.ds(base, IDXCHUNK)], dstbuf)
        pltpu.async_copy(table_hbm.at[srcbuf.at[0]], rows0, sem0)

        @pl.loop(0, IDXCHUNK, step=2)
        def _edge(j):
            pltpu.async_copy(table_hbm.at[srcbuf.at[j + 1]], rows1, sem1)
            pltpu.make_async_copy(table_hbm.at[srcbuf.at[j]], rows0,
                                  sem0).wait()
            pltpu.sync_copy(rows0, acc.at[dstbuf.at[j]], add=True)

            @pl.when(j + 2 < IDXCHUNK)
            def _():
                pltpu.async_copy(table_hbm.at[srcbuf.at[j + 2]], rows0, sem0)

            pltpu.make_async_copy(table_hbm.at[srcbuf.at[j + 1]], rows1,
                                  sem1).wait()
            pltpu.sync_copy(rows1, acc.at[dstbuf.at[j + 1]], add=True)

    plsc.subcore_barrier()

    @pl.when(s < NTILES - 1)
    def _():
        pltpu.sync_copy(acc.at[pl.ds(RPT * s, RPT)],
                        part_hbm.at[c, pl.ds(RPT * s, RPT)])

    @pl.when(s == NTILES - 1)
    def _():
        tail = N - RPT * (NTILES - 1)
        pltpu.sync_copy(acc.at[pl.ds(RPT * (NTILES - 1), tail)],
                        part_hbm.at[c, pl.ds(RPT * (NTILES - 1), tail)])


_sc_agg = functools.partial(
    pl.kernel,
    out_type=jax.ShapeDtypeStruct((NSC, N, D), jnp.float32),
    mesh=plsc.VectorSubcoreMesh(core_axis_name="c", subcore_axis_name="s"),
    scratch_types=[
        pltpu.VMEM_SHARED((ACC_ROWS, D), jnp.float32),
        pltpu.VMEM((IDXCHUNK, STREAM), jnp.int32),
        pltpu.VMEM((IDXCHUNK, STREAM), jnp.int32),
        pltpu.VMEM((2, STREAM, D), jnp.float32),
        pltpu.SemaphoreType.DMA,
        pltpu.SemaphoreType.DMA,
    ],
)(_sc_agg_body)


ROWB = 2048
TCGRID = (N + ROWB - 1) // ROWB


def _tc_hidden_body(p01, xp, dinv, w1, b1, o):
    p = p01[...]
    t = dinv[...] * (p[0] + p[1] + xp[...])
    h = jnp.dot(t, w1[...], preferred_element_type=jnp.float32) + b1[...]
    o[...] = dinv[...] * jnp.maximum(h, 0.0)


def _tc_hidden(p01, xp, dinv, w1, b1):
    grid = (TCGRID,)
    row_spec = pl.BlockSpec((ROWB, D), lambda i: (i, 0))
    return pl.pallas_call(
        _tc_hidden_body,
        grid=grid,
        in_specs=[pl.BlockSpec((2, ROWB, D), lambda i: (0, i, 0)),
                  row_spec,
                  pl.BlockSpec((ROWB, 1), lambda i: (i, 0)),
                  pl.BlockSpec((D, D), lambda i: (0, 0)),
                  pl.BlockSpec((1, D), lambda i: (0, 0))],
        out_specs=row_spec,
        out_shape=jax.ShapeDtypeStruct((N, D), jnp.float32),
    )(p01, xp, dinv, w1, b1)


def _tc_heads_body(q01, hp, dinv, wmut, bmu, wlvt, blv, epst, z, mu, lv):
    q = q01[...]
    a = dinv[...] * (q[0] + q[1] + hp[...])
    nt = (((1,), (1,)), ((), ()))
    m = lax.dot_general(wmut[...], a, nt,
                        preferred_element_type=jnp.float32) + bmu[...]
    v = lax.dot_general(wlvt[...], a, nt,
                        preferred_element_type=jnp.float32) + blv[...]
    mu[...] = m
    lv[...] = v
    z[...] = m + jnp.exp(0.5 * v) * epst[...]


def _tc_heads(q01, hp, dinv, wmut, bmu, wlvt, blv, epst):
    grid = (TCGRID,)
    row_spec = pl.BlockSpec((ROWB, D), lambda i: (i, 0))
    z_spec = pl.BlockSpec((DZ, ROWB), lambda i: (0, i))
    w_spec = pl.BlockSpec((DZ, D), lambda i: (0, 0))
    b_spec = pl.BlockSpec((DZ, 1), lambda i: (0, 0))
    zshape = jax.ShapeDtypeStruct((DZ, N), jnp.float32)
    return pl.pallas_call(
        _tc_heads_body,
        grid=grid,
        in_specs=[pl.BlockSpec((2, ROWB, D), lambda i: (0, i, 0)),
                  row_spec,
                  pl.BlockSpec((ROWB, 1), lambda i: (i, 0)),
                  w_spec, b_spec, w_spec, b_spec, z_spec],
        out_specs=(z_spec, z_spec, z_spec),
        out_shape=(zshape, zshape, zshape),
    )(q01, hp, dinv, wmut, bmu, wlvt, blv, epst)


def kernel(x, edge_index, W1, b1, Wmu, bmu, Wlv, blv):
    pad = EPAD - E
    src_pad = jnp.arange(pad, dtype=jnp.int32)
    dst_pad = N + jnp.broadcast_to(
        jnp.arange(ACC_ROWS - N, dtype=jnp.int32),
        (pad // (ACC_ROWS - N), ACC_ROWS - N)).reshape(-1)
    ei3 = jnp.concatenate(
        [edge_index.astype(jnp.int32),
         jnp.stack([src_pad, dst_pad])], axis=1).reshape(2, NSTREAMS, STREAM)

    degp = _sc_deg(ei3)
    dinv, xp = _sc_prep(x, degp)
    dinv2d = dinv.reshape(N, 1)
    zrows = jnp.zeros((RPT, D), jnp.float32)

    p = _sc_agg(xp, ei3, zrows)
    hp = _tc_hidden(p, xp, dinv2d, W1, b1.reshape(1, D))

    q = _sc_agg(hp, ei3, zrows)
    epst = jax.random.normal(jax.random.key(42), (N, DZ), jnp.float32).T
    zt, mut, lvt = _tc_heads(q, hp, dinv2d,
                             Wmu.T, bmu.reshape(DZ, 1),
                             Wlv.T, blv.reshape(DZ, 1), epst)
    return zt.T, mut.T, lvt.T
